# baseline (speedup 1.0000x reference)
import functools

import jax
import jax.numpy as jnp
from jax import lax
from jax.experimental import pallas as pl
from jax.experimental.pallas import tpu as pltpu
from jax.experimental.pallas import tpu_sc as plsc

_NUM_CORES = 2
_NUM_SUBCORES = 16
_NW = _NUM_CORES * _NUM_SUBCORES
_CHUNK = 8
_NBUF = 12
_NG = 3


@functools.lru_cache(maxsize=None)
def _make_gather(n, d):
    per_w = n // _NW
    nchunk = per_w // _CHUNK
    mesh = plsc.VectorSubcoreMesh(core_axis_name="c", subcore_axis_name="s")

    @functools.partial(
        pl.kernel,
        out_type=jax.ShapeDtypeStruct((n, d), jnp.float32),
        mesh=mesh,
        scratch_types=[
            pltpu.VMEM((per_w,), jnp.int32),
            pltpu.VMEM((_NBUF, _CHUNK, d), jnp.float32),
            pltpu.SemaphoreType.DMA((_NBUF,)),
            pltpu.SemaphoreType.DMA((_NBUF,)),
        ],
    )
    def k(pos_hbm, table_hbm, out_hbm, idx_v, rows_v, gsem, osem):
        wid = lax.axis_index("s") * _NUM_CORES + lax.axis_index("c")
        base = wid * per_w
        pltpu.sync_copy(pos_hbm.at[pl.ds(base, per_w)], idx_v)

        def gather_start(c, b):
            pltpu.async_copy(
                table_hbm.at[idx_v.at[pl.ds(c * _CHUNK, _CHUNK)]],
                rows_v.at[b],
                gsem.at[b],
            )

        def gather_wait(b):
            pltpu.make_async_copy(
                table_hbm.at[idx_v.at[pl.ds(0, _CHUNK)]], rows_v.at[b], gsem.at[b]
            ).wait()

        def store_start(c, b):
            pltpu.async_copy(
                rows_v.at[b], out_hbm.at[pl.ds(base + c * _CHUNK, _CHUNK)], osem.at[b]
            )

        def store_wait(b):
            pltpu.make_async_copy(
                rows_v.at[b], out_hbm.at[pl.ds(base, _CHUNK)], osem.at[b]
            ).wait()

        gs = _NBUF // _NG
        nbatch = nchunk // gs
        assert (nbatch - (_NG - 1)) % _NG == 0

        def batch_gathers(kb, grp):
            for b in range(gs):
                gather_start(kb * gs + b, grp * gs + b)

        def batch_stores(kb, grp):
            for b in range(gs):
                gather_wait(grp * gs + b)
            for b in range(gs):
                store_start(kb * gs + b, grp * gs + b)
            for b in range(gs):
                store_wait(grp * gs + b)

        for g in range(_NG - 1):
            batch_gathers(g, g)

        @pl.loop(0, nbatch - (_NG - 1), step=_NG)
        def _outer(kb):
            for g in range(_NG):
                batch_gathers(kb + g + _NG - 1, (g + _NG - 1) % _NG)
                batch_stores(kb + g, g)

        for g in range(_NG - 1):
            batch_stores(nbatch - (_NG - 1) + g, g)

    return k


def kernel(positions, table):
    b, s = positions.shape
    n = b * s
    d = table.shape[1]
    flat = positions.reshape(n).astype(jnp.int32)
    out = _make_gather(n, d)(flat, table)
    return out.reshape(b, s, d)


# speedup vs baseline: 1.0041x; 1.0041x over previous
import functools

import jax
import jax.numpy as jnp
from jax import lax
from jax.experimental import pallas as pl
from jax.experimental.pallas import tpu as pltpu
from jax.experimental.pallas import tpu_sc as plsc

_NUM_CORES = 2
_NUM_SUBCORES = 16
_NW = _NUM_CORES * _NUM_SUBCORES
_CHUNK = 8
_NBUF = 12
_NG = 3


@functools.lru_cache(maxsize=None)
def _make_gather(n, d):
    per_w = n // _NW
    nchunk = per_w // _CHUNK
    mesh = plsc.VectorSubcoreMesh(core_axis_name="c", subcore_axis_name="s")

    @functools.partial(
        pl.kernel,
        out_type=jax.ShapeDtypeStruct((n, d), jnp.float32),
        mesh=mesh,
        scratch_types=[
            pltpu.VMEM((per_w,), jnp.int32),
            pltpu.VMEM((_NG, (_NBUF // _NG) * _CHUNK, d), jnp.float32),
            pltpu.SemaphoreType.DMA((_NBUF,)),
            pltpu.SemaphoreType.DMA((_NG,)),
        ],
    )
    def k(pos_hbm, table_hbm, out_hbm, idx_v, rows_v, gsem, osem):
        wid = lax.axis_index("s") * _NUM_CORES + lax.axis_index("c")
        base = wid * per_w
        pltpu.sync_copy(pos_hbm.at[pl.ds(base, per_w)], idx_v)

        gs = _NBUF // _NG
        nbatch = nchunk // gs
        assert (nbatch - (_NG - 1)) % _NG == 0

        def gather_start(c, grp, b):
            pltpu.async_copy(
                table_hbm.at[idx_v.at[pl.ds(c * _CHUNK, _CHUNK)]],
                rows_v.at[grp, pl.ds(b * _CHUNK, _CHUNK)],
                gsem.at[grp * gs + b],
            )

        def gather_wait(grp, b):
            pltpu.make_async_copy(
                table_hbm.at[idx_v.at[pl.ds(0, _CHUNK)]],
                rows_v.at[grp, pl.ds(b * _CHUNK, _CHUNK)],
                gsem.at[grp * gs + b],
            ).wait()

        def batch_gathers(kb, grp):
            for b in range(gs):
                gather_start(kb * gs + b, grp, b)

        def batch_stores(kb, grp):
            for b in range(gs):
                gather_wait(grp, b)
            pltpu.async_copy(
                rows_v.at[grp],
                out_hbm.at[pl.ds(base + kb * gs * _CHUNK, gs * _CHUNK)],
                osem.at[grp],
            )
            pltpu.make_async_copy(
                rows_v.at[grp],
                out_hbm.at[pl.ds(base, gs * _CHUNK)],
                osem.at[grp],
            ).wait()

        for g in range(_NG - 1):
            batch_gathers(g, g)

        @pl.loop(0, nbatch - (_NG - 1), step=_NG)
        def _outer(kb):
            for g in range(_NG):
                batch_gathers(kb + g + _NG - 1, (g + _NG - 1) % _NG)
                batch_stores(kb + g, g)

        for g in range(_NG - 1):
            batch_stores(nbatch - (_NG - 1) + g, g)

    return k


def kernel(positions, table):
    b, s = positions.shape
    n = b * s
    d = table.shape[1]
    flat = positions.reshape(n).astype(jnp.int32)
    out = _make_gather(n, d)(flat, table)
    return out.reshape(b, s, d)
